# baseline (speedup 1.0000x reference)
import functools

import jax
import jax.numpy as jnp
from jax import lax
from jax.experimental import pallas as pl
from jax.experimental.pallas import tpu as pltpu
from jax.experimental.pallas import tpu_sc as plsc

_NC = 2
_NS = 16
_NW = _NC * _NS


def _field_embedding_lookup(x_flat, W_flat, *, total, V, E, F):
    rows_per_w = total // _NW
    CH = 1664
    n_ch = rows_per_w // CH
    n_sl = CH // 16

    mesh = plsc.VectorSubcoreMesh(
        core_axis_name="c", subcore_axis_name="s",
        num_cores=_NC, num_subcores=_NS)

    @functools.partial(
        pl.kernel,
        out_type=jax.ShapeDtypeStruct((total, E), jnp.float32),
        mesh=mesh,
        scratch_types=[
            pltpu.VMEM((CH,), jnp.int32),
            pltpu.VMEM((CH,), jnp.int32),
            pltpu.VMEM((CH, E), jnp.float32),
            pltpu.SemaphoreType.DMA,
        ],
        compiler_params=pltpu.CompilerParams(use_tc_tiling_on_sc=False),
    )
    def k(idx_hbm, table_hbm, out_hbm, pat_v, idx_v, rows_v, sem):
        wid = lax.axis_index("s") * _NC + lax.axis_index("c")
        base = wid * rows_per_w

        def pat_body(i, _):
            pos = i * 16 + lax.iota(jnp.int32, 16)
            pat_v[pl.ds(i * 16, 16)] = lax.rem(pos, F) * V
            return ()
        lax.fori_loop(0, n_sl, pat_body, ())

        def chunk_body(c, _):
            off = base + c * CH
            pltpu.sync_copy(idx_hbm.at[pl.ds(off, CH)], idx_v)

            def add_body(i, _):
                s = pl.ds(i * 16, 16)
                idx_v[s] = idx_v[s] + pat_v[s]
                return ()
            lax.fori_loop(0, n_sl, add_body, ())

            pltpu.async_copy(table_hbm.at[idx_v], rows_v, sem).wait()
            pltpu.sync_copy(rows_v, out_hbm.at[pl.ds(off, CH)])
            return ()
        lax.fori_loop(0, n_ch, chunk_body, ())

    return k(x_flat, W_flat)


def kernel(x, W):
    B, F = x.shape
    _, V, E = W.shape
    W_flat = W.reshape(F * V, E)
    x_flat = x.reshape(B * F)
    y = _field_embedding_lookup(x_flat, W_flat, total=B * F, V=V, E=E, F=F)
    return y.reshape(B, F, E)
